# baseline (speedup 1.0000x reference)
import functools

import jax
import jax.numpy as jnp
from jax import lax
from jax.experimental import pallas as pl
from jax.experimental.pallas import tpu as pltpu
from jax.experimental.pallas import tpu_sc as plsc

N = 10000
E = 320000
B = 2048
D_IN = 128
D_HID = 256
D_OUT = 128

NC = 2
NS = 16
NW = NC * NS

NPAD = 10240
RB = 1024

ROWW = NPAD
A_SIZE = N * ROWW
A_PER_CORE = A_SIZE // NC
A_PER_TILE = A_PER_CORE // NS
A_CHUNK = 12800
A_ZITERS = A_PER_TILE // A_CHUNK

EK = 80
E_PER_TILE_AGG = E // NW
E_ITERS_AGG = E_PER_TILE_AGG // EK
E_PER_TILE_A = E // NS
E_ITERS_A = E_PER_TILE_A // EK

Q_PER_TILE = B // NW

_mesh = functools.partial(
    plsc.VectorSubcoreMesh,
    core_axis_name="c", subcore_axis_name="s", num_cores=NC, num_subcores=NS)


def _sds(shape, dtype=jnp.float32):
    return jax.ShapeDtypeStruct(shape, dtype)


@functools.partial(
    pl.kernel,
    out_type=(_sds((NC, NPAD, D_IN)), _sds((NC, NPAD))),
    mesh=_mesh(),
    scratch_types=[
        pltpu.VMEM_SHARED((NPAD, D_IN), jnp.float32),
        pltpu.VMEM_SHARED((NPAD,), jnp.float32),
        pltpu.VMEM((EK,), jnp.int32),
        pltpu.VMEM((EK,), jnp.int32),
        pltpu.VMEM((EK, D_IN), jnp.float32),
        pltpu.VMEM((128, D_IN), jnp.float32),
        pltpu.VMEM((1024,), jnp.float32),
        pltpu.VMEM((EK,), jnp.float32),
        pltpu.SemaphoreType.DMA,
    ],
)
def _sc_agg(x_hbm, row_hbm, col_hbm, zb_hbm, zd_hbm, on_hbm,
            agg_out, deg_out,
            agg_sh, deg_sh, ridx, cidx, rows, zb_v, zd_v, on_v, sem):
    c = lax.axis_index("c")
    s = lax.axis_index("s")
    wid = s * NC + c
    pltpu.sync_copy(zb_hbm, zb_v)
    pltpu.sync_copy(zd_hbm, zd_v)
    pltpu.sync_copy(on_hbm, on_v)
    for k in range(5):
        pltpu.sync_copy(zb_v, agg_sh.at[pl.ds(s * 640 + k * 128, 128)])

    @pl.when(s < 10)
    def _():
        pltpu.sync_copy(zd_v, deg_sh.at[pl.ds(s * 1024, 1024)])

    plsc.subcore_barrier()
    base = wid * E_PER_TILE_AGG

    def step(i, carry):
        off = base + i * EK
        pltpu.sync_copy(row_hbm.at[pl.ds(off, EK)], ridx)
        pltpu.sync_copy(col_hbm.at[pl.ds(off, EK)], cidx)
        pltpu.async_copy(x_hbm.at[cidx], rows, sem).wait()
        pltpu.sync_copy(rows, agg_sh.at[ridx], add=True)
        pltpu.sync_copy(on_v, deg_sh.at[ridx], add=True)
        return carry

    lax.fori_loop(0, E_ITERS_AGG, step, 0)
    plsc.subcore_barrier()
    pltpu.sync_copy(agg_sh.at[pl.ds(s * 640, 640)],
                    agg_out.at[c, pl.ds(s * 640, 640)])

    @pl.when(s == 0)
    def _():
        pltpu.sync_copy(deg_sh, deg_out.at[c])


@functools.partial(
    pl.kernel,
    out_type=_sds((A_SIZE,)),
    mesh=_mesh(),
    scratch_types=[
        pltpu.VMEM((EK,), jnp.int32),
        pltpu.VMEM((EK,), jnp.int32),
        pltpu.VMEM((EK,), jnp.int32),
        pltpu.VMEM((A_CHUNK,), jnp.float32),
        pltpu.VMEM((EK,), jnp.float32),
        pltpu.SemaphoreType.DMA,
    ],
)
def _sc_abuild(row_hbm, col_hbm, zf_hbm, on_hbm,
               a_out, ridx, cidx, aidx, zf_v, on_v, sem):
    c = lax.axis_index("c")
    s = lax.axis_index("s")
    pltpu.sync_copy(zf_hbm, zf_v)
    pltpu.sync_copy(on_hbm, on_v)
    zbase = c * A_PER_CORE + s * A_PER_TILE

    def zstep(i, carry):
        pltpu.sync_copy(zf_v, a_out.at[pl.ds(zbase + i * A_CHUNK, A_CHUNK)])
        return carry

    lax.fori_loop(0, A_ZITERS, zstep, 0)
    plsc.subcore_barrier()
    lo = c * (N // NC)
    hi = lo + (N // NC)
    sink = lo * ROWW + N
    ebase = s * E_PER_TILE_A

    def step(i, carry):
        off = ebase + i * EK
        pltpu.sync_copy(row_hbm.at[pl.ds(off, EK)], ridx)
        pltpu.sync_copy(col_hbm.at[pl.ds(off, EK)], cidx)
        for j in range(EK // 16):
            r = ridx[pl.ds(j * 16, 16)]
            cc = cidx[pl.ds(j * 16, 16)]
            owned = (r >= lo) & (r < hi)
            aidx[pl.ds(j * 16, 16)] = jnp.where(owned, r * ROWW + cc, sink)
        pltpu.async_copy(on_v, a_out.at[aidx], sem).wait()
        return carry

    lax.fori_loop(0, E_ITERS_A, step, 0)


@functools.partial(
    pl.kernel,
    out_type=(_sds((B, D_OUT)), _sds((B, D_OUT)),
              _sds((B, NPAD // 128, 128))),
    mesh=_mesh(),
    scratch_types=[
        pltpu.VMEM((Q_PER_TILE,), jnp.int32),
        pltpu.VMEM((Q_PER_TILE,), jnp.int32),
        pltpu.VMEM((2 * Q_PER_TILE,), jnp.int32),
        pltpu.VMEM((Q_PER_TILE, D_OUT), jnp.float32),
        pltpu.VMEM((8, ROWW), jnp.float32),
        pltpu.VMEM((NPAD // 128, 128), jnp.float32),
        pltpu.SemaphoreType.DMA,
    ],
)
def _sc_gm(hn_hbm, a2_hbm, u_hbm, v_hbm, uvi_hbm,
           hu_out, hv_out, m_out,
           uq, vq, uvq, hnr, auv, mrow, sem):
    c = lax.axis_index("c")
    s = lax.axis_index("s")
    wid = s * NC + c
    qb = wid * Q_PER_TILE
    pltpu.sync_copy(u_hbm.at[pl.ds(qb, Q_PER_TILE)], uq)
    pltpu.async_copy(hn_hbm.at[uq], hnr, sem).wait()
    pltpu.sync_copy(hnr, hu_out.at[pl.ds(qb, Q_PER_TILE)])
    pltpu.sync_copy(v_hbm.at[pl.ds(qb, Q_PER_TILE)], vq)
    pltpu.async_copy(hn_hbm.at[vq], hnr, sem).wait()
    pltpu.sync_copy(hnr, hv_out.at[pl.ds(qb, Q_PER_TILE)])
    pltpu.sync_copy(uvi_hbm.at[pl.ds(2 * qb, 2 * Q_PER_TILE)], uvq)
    z16 = jnp.zeros((16,), jnp.float32)
    for j in range(N // 16, NPAD // 16):
        mrow[j // 8, pl.ds((j % 8) * 16, 16)] = z16

    def qstep(g, carry):
        pltpu.async_copy(a2_hbm.at[uvq.at[pl.ds(8 * g, 8)]], auv, sem).wait()
        for i in range(4):
            def colstep(j, carry2, _i=i):
                st = j * 16
                mrow[j // 8, pl.ds((j % 8) * 16, 16)] = (
                    auv[2 * _i, pl.ds(st, 16)] *
                    auv[2 * _i + 1, pl.ds(st, 16)])
                return carry2

            lax.fori_loop(0, N // 16, colstep, 0)
            pltpu.sync_copy(mrow, m_out.at[qb + 4 * g + i])
        return carry

    lax.fori_loop(0, Q_PER_TILE // 4, qstep, 0)


def _tc_mlp_body(x_ref, agg_ref, deg_ref, w1_ref, b1_ref, w2_ref, b2_ref,
                 w3_ref, b3_ref, hn_ref):
    deg = deg_ref[0] + deg_ref[1] + 1e-6
    agg = agg_ref[0] + agg_ref[1]
    h = x_ref[...] + agg / deg
    h = jnp.maximum(jnp.dot(h, w1_ref[...],
                            preferred_element_type=jnp.float32)
                    + b1_ref[...], 0.0)
    h = jnp.maximum(jnp.dot(h, w2_ref[...],
                            preferred_element_type=jnp.float32)
                    + b2_ref[...], 0.0)
    h = jnp.dot(h, w3_ref[...], preferred_element_type=jnp.float32) \
        + b3_ref[...]
    nrm = jnp.sqrt(jnp.sum(h * h, axis=1, keepdims=True))
    hn_ref[...] = h / jnp.maximum(nrm, 1e-8)


def _tc_mlp(xp, agg2, deg3, W1, b1, W2, b2, W3, b3):
    g = NPAD // RB
    return pl.pallas_call(
        _tc_mlp_body,
        grid=(g,),
        in_specs=[
            pl.BlockSpec((RB, D_IN), lambda i: (i, 0)),
            pl.BlockSpec((NC, RB, D_IN), lambda i: (0, i, 0)),
            pl.BlockSpec((NC, RB, 1), lambda i: (0, i, 0)),
            pl.BlockSpec((D_IN, D_HID), lambda i: (0, 0)),
            pl.BlockSpec((1, D_HID), lambda i: (0, 0)),
            pl.BlockSpec((D_HID, D_HID), lambda i: (0, 0)),
            pl.BlockSpec((1, D_HID), lambda i: (0, 0)),
            pl.BlockSpec((D_HID, D_OUT), lambda i: (0, 0)),
            pl.BlockSpec((1, D_OUT), lambda i: (0, 0)),
        ],
        out_specs=pl.BlockSpec((RB, D_OUT), lambda i: (i, 0)),
        out_shape=_sds((NPAD, D_OUT)),
    )(xp, agg2, deg3, W1, b1, W2, b2, W3, b3)


def _tc_query_body(hu_ref, hv_ref, hn_ref, m_ref, w_ref, acc):
    i = pl.program_id(0)

    @pl.when(i == 0)
    def _():
        acc[...] = jnp.zeros_like(acc)

    hnb = hn_ref[...]
    cl = lax.dot_general(hu_ref[...], hnb, (((1,), (1,)), ((), ())),
                         preferred_element_type=jnp.float32)
    cr = lax.dot_general(hv_ref[...], hnb, (((1,), (1,)), ((), ())),
                         preferred_element_type=jnp.float32)
    acc[...] += jnp.sum(m_ref[...] * cl * cr, axis=1, keepdims=True)

    @pl.when(i == pl.num_programs(0) - 1)
    def _():
        w_ref[...] = jax.nn.sigmoid(acc[...])


def _tc_query(hu, hv, hn, M):
    g = NPAD // RB
    return pl.pallas_call(
        _tc_query_body,
        grid=(g,),
        in_specs=[
            pl.BlockSpec((B, D_OUT), lambda i: (0, 0)),
            pl.BlockSpec((B, D_OUT), lambda i: (0, 0)),
            pl.BlockSpec((RB, D_OUT), lambda i: (i, 0)),
            pl.BlockSpec((B, RB), lambda i: (0, i)),
        ],
        out_specs=pl.BlockSpec((B, 1), lambda i: (0, 0)),
        out_shape=_sds((B, 1)),
        scratch_shapes=[pltpu.VMEM((B, 1), jnp.float32)],
    )(hu, hv, hn, M)


def kernel(x, edges, adj, W1, b1, W2, b2, W3, b3):
    row = adj[0].astype(jnp.int32)
    col = adj[1].astype(jnp.int32)
    u = edges[0].astype(jnp.int32)
    v = edges[1].astype(jnp.int32)

    zb = jnp.zeros((128, D_IN), jnp.float32)
    zd = jnp.zeros((1024,), jnp.float32)
    on = jnp.ones((EK,), jnp.float32)
    zf = jnp.zeros((A_CHUNK,), jnp.float32)
    uvi = jnp.stack([u, v], axis=1).reshape(-1)

    agg2, deg2 = _sc_agg(x, row, col, zb, zd, on)
    aflat = _sc_abuild(row, col, zf, on)
    xp = jnp.pad(x, ((0, NPAD - N), (0, 0)))
    deg3 = deg2.reshape(NC, NPAD, 1)
    hn = _tc_mlp(xp, agg2, deg3, W1, b1.reshape(1, D_HID),
                 W2, b2.reshape(1, D_HID), W3, b3.reshape(1, D_OUT))
    a2 = aflat.reshape(N, ROWW)
    hu, hv, M3 = _sc_gm(hn, a2, u, v, uvi)
    w = _tc_query(hu, hv, hn, M3.reshape(B, NPAD))
    return w.reshape(B)


# speedup vs baseline: 7.6766x; 7.6766x over previous
import functools

import jax
import jax.numpy as jnp
from jax import lax
from jax.experimental import pallas as pl
from jax.experimental.pallas import tpu as pltpu
from jax.experimental.pallas import tpu_sc as plsc

N = 10000
E = 320000
B = 2048
D_IN = 128
D_HID = 256
D_OUT = 128

NC = 2
NS = 16
NW = NC * NS

NPAD = 10240
RB = 1024

ROWW = NPAD
A_SIZE = N * ROWW
A_PER_CORE = A_SIZE // NC
A_PER_TILE = A_PER_CORE // NS
A_CHUNK = 12800
A_ZITERS = A_PER_TILE // A_CHUNK

EK = 80
E_PER_TILE_AGG = E // NW
E_ITERS_AGG = E_PER_TILE_AGG // EK
E_PER_TILE_A = E // NS
E_ITERS_A = E_PER_TILE_A // EK

Q_PER_TILE = B // NW

_mesh = functools.partial(
    plsc.VectorSubcoreMesh,
    core_axis_name="c", subcore_axis_name="s", num_cores=NC, num_subcores=NS)


def _sds(shape, dtype=jnp.float32):
    return jax.ShapeDtypeStruct(shape, dtype)


@functools.partial(
    pl.kernel,
    out_type=(_sds((NC, NPAD, D_IN)), _sds((NC, NPAD))),
    mesh=_mesh(),
    scratch_types=[
        pltpu.VMEM_SHARED((NPAD, D_IN), jnp.float32),
        pltpu.VMEM_SHARED((NPAD,), jnp.float32),
        pltpu.VMEM((EK,), jnp.int32),
        pltpu.VMEM((EK,), jnp.int32),
        pltpu.VMEM((EK, D_IN), jnp.float32),
        pltpu.VMEM((128, D_IN), jnp.float32),
        pltpu.VMEM((1024,), jnp.float32),
        pltpu.VMEM((EK,), jnp.float32),
        pltpu.SemaphoreType.DMA,
    ],
)
def _sc_agg(x_hbm, row_hbm, col_hbm, zb_hbm, zd_hbm, on_hbm,
            agg_out, deg_out,
            agg_sh, deg_sh, ridx, cidx, rows, zb_v, zd_v, on_v, sem):
    c = lax.axis_index("c")
    s = lax.axis_index("s")
    wid = s * NC + c
    pltpu.sync_copy(zb_hbm, zb_v)
    pltpu.sync_copy(zd_hbm, zd_v)
    pltpu.sync_copy(on_hbm, on_v)
    for k in range(5):
        pltpu.sync_copy(zb_v, agg_sh.at[pl.ds(s * 640 + k * 128, 128)])

    @pl.when(s < 10)
    def _():
        pltpu.sync_copy(zd_v, deg_sh.at[pl.ds(s * 1024, 1024)])

    plsc.subcore_barrier()
    base = wid * E_PER_TILE_AGG

    def step(i, carry):
        off = base + i * EK
        pltpu.sync_copy(row_hbm.at[pl.ds(off, EK)], ridx)
        pltpu.sync_copy(col_hbm.at[pl.ds(off, EK)], cidx)
        pltpu.async_copy(x_hbm.at[cidx], rows, sem).wait()
        pltpu.sync_copy(rows, agg_sh.at[ridx], add=True)
        pltpu.sync_copy(on_v, deg_sh.at[ridx], add=True)
        return carry

    lax.fori_loop(0, E_ITERS_AGG, step, 0)
    plsc.subcore_barrier()
    pltpu.sync_copy(agg_sh.at[pl.ds(s * 640, 640)],
                    agg_out.at[c, pl.ds(s * 640, 640)])

    @pl.when(s == 0)
    def _():
        pltpu.sync_copy(deg_sh, deg_out.at[c])


@functools.partial(
    pl.kernel,
    out_type=_sds((A_SIZE,)),
    mesh=_mesh(),
    scratch_types=[
        pltpu.VMEM((EK,), jnp.int32),
        pltpu.VMEM((EK,), jnp.int32),
        pltpu.VMEM((EK,), jnp.int32),
        pltpu.VMEM((A_CHUNK,), jnp.float32),
        pltpu.VMEM((EK,), jnp.float32),
        pltpu.SemaphoreType.DMA,
    ],
)
def _sc_abuild(row_hbm, col_hbm, zf_hbm, on_hbm,
               a_out, ridx, cidx, aidx, zf_v, on_v, sem):
    c = lax.axis_index("c")
    s = lax.axis_index("s")
    pltpu.sync_copy(zf_hbm, zf_v)
    pltpu.sync_copy(on_hbm, on_v)
    zbase = c * A_PER_CORE + s * A_PER_TILE

    def zstep(i, carry):
        pltpu.sync_copy(zf_v, a_out.at[pl.ds(zbase + i * A_CHUNK, A_CHUNK)])
        return carry

    lax.fori_loop(0, A_ZITERS, zstep, 0)
    plsc.subcore_barrier()
    lo = c * (N // NC)
    hi = lo + (N // NC)
    sink0 = (lo + s) * ROWW + N + lax.iota(jnp.int32, 16)
    ebase = s * E_PER_TILE_A

    def step(i, carry):
        off = ebase + i * EK
        pltpu.sync_copy(row_hbm.at[pl.ds(off, EK)], ridx)
        pltpu.sync_copy(col_hbm.at[pl.ds(off, EK)], cidx)
        sink = sink0 + lax.rem(i, 15) * 16
        for j in range(EK // 16):
            r = ridx[pl.ds(j * 16, 16)]
            cc = cidx[pl.ds(j * 16, 16)]
            owned = (r >= lo) & (r < hi)
            aidx[pl.ds(j * 16, 16)] = jnp.where(owned, r * ROWW + cc, sink)
        pltpu.async_copy(on_v, a_out.at[aidx], sem).wait()
        return carry

    lax.fori_loop(0, E_ITERS_A, step, 0)


@functools.partial(
    pl.kernel,
    out_type=(_sds((B, D_OUT)), _sds((B, D_OUT)),
              _sds((B, NPAD // 128, 128))),
    mesh=_mesh(),
    scratch_types=[
        pltpu.VMEM((Q_PER_TILE,), jnp.int32),
        pltpu.VMEM((Q_PER_TILE,), jnp.int32),
        pltpu.VMEM((2 * Q_PER_TILE,), jnp.int32),
        pltpu.VMEM((Q_PER_TILE, D_OUT), jnp.float32),
        pltpu.VMEM((8, ROWW), jnp.float32),
        pltpu.VMEM((NPAD // 128, 128), jnp.float32),
        pltpu.SemaphoreType.DMA,
    ],
)
def _sc_gm(hn_hbm, a2_hbm, u_hbm, v_hbm, uvi_hbm,
           hu_out, hv_out, m_out,
           uq, vq, uvq, hnr, auv, mrow, sem):
    c = lax.axis_index("c")
    s = lax.axis_index("s")
    wid = s * NC + c
    qb = wid * Q_PER_TILE
    pltpu.sync_copy(u_hbm.at[pl.ds(qb, Q_PER_TILE)], uq)
    pltpu.async_copy(hn_hbm.at[uq], hnr, sem).wait()
    pltpu.sync_copy(hnr, hu_out.at[pl.ds(qb, Q_PER_TILE)])
    pltpu.sync_copy(v_hbm.at[pl.ds(qb, Q_PER_TILE)], vq)
    pltpu.async_copy(hn_hbm.at[vq], hnr, sem).wait()
    pltpu.sync_copy(hnr, hv_out.at[pl.ds(qb, Q_PER_TILE)])
    pltpu.sync_copy(uvi_hbm.at[pl.ds(2 * qb, 2 * Q_PER_TILE)], uvq)
    z16 = jnp.zeros((16,), jnp.float32)
    for j in range(N // 16, NPAD // 16):
        mrow[j // 8, pl.ds((j % 8) * 16, 16)] = z16

    def qstep(g, carry):
        pltpu.async_copy(a2_hbm.at[uvq.at[pl.ds(8 * g, 8)]], auv, sem).wait()
        for i in range(4):
            def colstep(j, carry2, _i=i):
                st = j * 16
                mrow[j // 8, pl.ds((j % 8) * 16, 16)] = (
                    auv[2 * _i, pl.ds(st, 16)] *
                    auv[2 * _i + 1, pl.ds(st, 16)])
                return carry2

            lax.fori_loop(0, N // 16, colstep, 0)
            pltpu.sync_copy(mrow, m_out.at[qb + 4 * g + i])
        return carry

    lax.fori_loop(0, Q_PER_TILE // 4, qstep, 0)


def _tc_mlp_body(x_ref, agg_ref, deg_ref, w1_ref, b1_ref, w2_ref, b2_ref,
                 w3_ref, b3_ref, hn_ref):
    deg = deg_ref[0] + deg_ref[1] + 1e-6
    agg = agg_ref[0] + agg_ref[1]
    h = x_ref[...] + agg / deg
    h = jnp.maximum(jnp.dot(h, w1_ref[...],
                            preferred_element_type=jnp.float32)
                    + b1_ref[...], 0.0)
    h = jnp.maximum(jnp.dot(h, w2_ref[...],
                            preferred_element_type=jnp.float32)
                    + b2_ref[...], 0.0)
    h = jnp.dot(h, w3_ref[...], preferred_element_type=jnp.float32) \
        + b3_ref[...]
    nrm = jnp.sqrt(jnp.sum(h * h, axis=1, keepdims=True))
    hn_ref[...] = h / jnp.maximum(nrm, 1e-8)


def _tc_mlp(xp, agg2, deg3, W1, b1, W2, b2, W3, b3):
    g = NPAD // RB
    return pl.pallas_call(
        _tc_mlp_body,
        grid=(g,),
        in_specs=[
            pl.BlockSpec((RB, D_IN), lambda i: (i, 0)),
            pl.BlockSpec((NC, RB, D_IN), lambda i: (0, i, 0)),
            pl.BlockSpec((NC, RB, 1), lambda i: (0, i, 0)),
            pl.BlockSpec((D_IN, D_HID), lambda i: (0, 0)),
            pl.BlockSpec((1, D_HID), lambda i: (0, 0)),
            pl.BlockSpec((D_HID, D_HID), lambda i: (0, 0)),
            pl.BlockSpec((1, D_HID), lambda i: (0, 0)),
            pl.BlockSpec((D_HID, D_OUT), lambda i: (0, 0)),
            pl.BlockSpec((1, D_OUT), lambda i: (0, 0)),
        ],
        out_specs=pl.BlockSpec((RB, D_OUT), lambda i: (i, 0)),
        out_shape=_sds((NPAD, D_OUT)),
    )(xp, agg2, deg3, W1, b1, W2, b2, W3, b3)


def _tc_query_body(hu_ref, hv_ref, hn_ref, m_ref, w_ref, acc):
    i = pl.program_id(0)

    @pl.when(i == 0)
    def _():
        acc[...] = jnp.zeros_like(acc)

    hnb = hn_ref[...]
    cl = lax.dot_general(hu_ref[...], hnb, (((1,), (1,)), ((), ())),
                         preferred_element_type=jnp.float32)
    cr = lax.dot_general(hv_ref[...], hnb, (((1,), (1,)), ((), ())),
                         preferred_element_type=jnp.float32)
    acc[...] += jnp.sum(m_ref[...] * cl * cr, axis=1, keepdims=True)

    @pl.when(i == pl.num_programs(0) - 1)
    def _():
        w_ref[...] = jax.nn.sigmoid(acc[...])


def _tc_query(hu, hv, hn, M):
    g = NPAD // RB
    return pl.pallas_call(
        _tc_query_body,
        grid=(g,),
        in_specs=[
            pl.BlockSpec((B, D_OUT), lambda i: (0, 0)),
            pl.BlockSpec((B, D_OUT), lambda i: (0, 0)),
            pl.BlockSpec((RB, D_OUT), lambda i: (i, 0)),
            pl.BlockSpec((B, RB), lambda i: (0, i)),
        ],
        out_specs=pl.BlockSpec((B, 1), lambda i: (0, 0)),
        out_shape=_sds((B, 1)),
        scratch_shapes=[pltpu.VMEM((B, 1), jnp.float32)],
    )(hu, hv, hn, M)


def kernel(x, edges, adj, W1, b1, W2, b2, W3, b3):
    row = adj[0].astype(jnp.int32)
    col = adj[1].astype(jnp.int32)
    u = edges[0].astype(jnp.int32)
    v = edges[1].astype(jnp.int32)

    zb = jnp.zeros((128, D_IN), jnp.float32)
    zd = jnp.zeros((1024,), jnp.float32)
    on = jnp.ones((EK,), jnp.float32)
    zf = jnp.zeros((A_CHUNK,), jnp.float32)
    uvi = jnp.stack([u, v], axis=1).reshape(-1)

    agg2, deg2 = _sc_agg(x, row, col, zb, zd, on)
    aflat = _sc_abuild(row, col, zf, on)
    xp = jnp.pad(x, ((0, NPAD - N), (0, 0)))
    deg3 = deg2.reshape(NC, NPAD, 1)
    hn = _tc_mlp(xp, agg2, deg3, W1, b1.reshape(1, D_HID),
                 W2, b2.reshape(1, D_HID), W3, b3.reshape(1, D_OUT))
    a2 = aflat.reshape(N, ROWW)
    hu, hv, M3 = _sc_gm(hn, a2, u, v, uvi)
    w = _tc_query(hu, hv, hn, M3.reshape(B, NPAD))
    return w.reshape(B)


# speedup vs baseline: 8.0634x; 1.0504x over previous
import functools

import jax
import jax.numpy as jnp
from jax import lax
from jax.experimental import pallas as pl
from jax.experimental.pallas import tpu as pltpu
from jax.experimental.pallas import tpu_sc as plsc

N = 10000
E = 320000
B = 2048
D_IN = 128
D_HID = 256
D_OUT = 128

NC = 2
NS = 16
NW = NC * NS

NPAD = 10240
RB = 1024

ROWW = NPAD
A_SIZE = N * ROWW
A_PER_CORE = A_SIZE // NC
A_PER_TILE = A_PER_CORE // NS
A_CHUNK = 64000
A_ZITERS = A_PER_TILE // A_CHUNK

EK = 80
E_PER_TILE_AGG = E // NW
E_ITERS_AGG = E_PER_TILE_AGG // EK
E_PER_TILE_A = E // NS
E_ITERS_A = E_PER_TILE_A // EK

Q_PER_TILE = B // NW

_mesh = functools.partial(
    plsc.VectorSubcoreMesh,
    core_axis_name="c", subcore_axis_name="s", num_cores=NC, num_subcores=NS)


def _sds(shape, dtype=jnp.float32):
    return jax.ShapeDtypeStruct(shape, dtype)


@functools.partial(
    pl.kernel,
    out_type=(_sds((NC, NPAD, D_IN)), _sds((NC, NPAD))),
    mesh=_mesh(),
    scratch_types=[
        pltpu.VMEM_SHARED((NPAD, D_IN), jnp.float32),
        pltpu.VMEM_SHARED((NPAD,), jnp.float32),
        pltpu.VMEM((EK,), jnp.int32),
        pltpu.VMEM((EK,), jnp.int32),
        pltpu.VMEM((EK, D_IN), jnp.float32),
        pltpu.VMEM((128, D_IN), jnp.float32),
        pltpu.VMEM((1024,), jnp.float32),
        pltpu.VMEM((EK,), jnp.float32),
        pltpu.SemaphoreType.DMA,
    ],
)
def _sc_agg(x_hbm, row_hbm, col_hbm, zb_hbm, zd_hbm, on_hbm,
            agg_out, deg_out,
            agg_sh, deg_sh, ridx, cidx, rows, zb_v, zd_v, on_v, sem):
    c = lax.axis_index("c")
    s = lax.axis_index("s")
    wid = s * NC + c
    pltpu.sync_copy(zb_hbm, zb_v)
    pltpu.sync_copy(zd_hbm, zd_v)
    pltpu.sync_copy(on_hbm, on_v)
    for k in range(5):
        pltpu.sync_copy(zb_v, agg_sh.at[pl.ds(s * 640 + k * 128, 128)])

    @pl.when(s < 10)
    def _():
        pltpu.sync_copy(zd_v, deg_sh.at[pl.ds(s * 1024, 1024)])

    plsc.subcore_barrier()
    base = wid * E_PER_TILE_AGG

    def step(i, carry):
        off = base + i * EK
        pltpu.sync_copy(row_hbm.at[pl.ds(off, EK)], ridx)
        pltpu.sync_copy(col_hbm.at[pl.ds(off, EK)], cidx)
        pltpu.async_copy(x_hbm.at[cidx], rows, sem).wait()
        pltpu.sync_copy(rows, agg_sh.at[ridx], add=True)
        pltpu.sync_copy(on_v, deg_sh.at[ridx], add=True)
        return carry

    lax.fori_loop(0, E_ITERS_AGG, step, 0)
    plsc.subcore_barrier()
    pltpu.sync_copy(agg_sh.at[pl.ds(s * 640, 640)],
                    agg_out.at[c, pl.ds(s * 640, 640)])

    @pl.when(s == 0)
    def _():
        pltpu.sync_copy(deg_sh, deg_out.at[c])


@functools.partial(
    pl.kernel,
    out_type=_sds((A_SIZE,)),
    mesh=_mesh(),
    scratch_types=[
        pltpu.VMEM((E_PER_TILE_A,), jnp.int32),
        pltpu.VMEM((E_PER_TILE_A,), jnp.int32),
        pltpu.VMEM((EK,), jnp.int32),
        pltpu.VMEM((EK,), jnp.int32),
        pltpu.VMEM((A_CHUNK,), jnp.float32),
        pltpu.VMEM((EK,), jnp.float32),
        pltpu.SemaphoreType.DMA,
        pltpu.SemaphoreType.DMA,
        pltpu.SemaphoreType.DMA,
    ],
)
def _sc_abuild(row_hbm, col_hbm, zf_hbm, on_hbm,
               a_out, ridx_all, cidx_all, aidx0, aidx1, zf_v, on_v,
               zsem, sem0, sem1):
    c = lax.axis_index("c")
    s = lax.axis_index("s")
    ebase = s * E_PER_TILE_A
    pltpu.sync_copy(zf_hbm, zf_v)
    pltpu.sync_copy(on_hbm, on_v)
    pltpu.sync_copy(row_hbm.at[pl.ds(ebase, E_PER_TILE_A)], ridx_all)
    pltpu.sync_copy(col_hbm.at[pl.ds(ebase, E_PER_TILE_A)], cidx_all)
    zbase = c * A_PER_CORE + s * A_PER_TILE

    def zstep(i, carry):
        pltpu.async_copy(zf_v, a_out.at[pl.ds(zbase + i * A_CHUNK, A_CHUNK)],
                         zsem)
        return carry

    lax.fori_loop(0, A_ZITERS, zstep, 0)

    def zdrain(i, carry):
        pltpu.make_async_copy(
            zf_v, a_out.at[pl.ds(zbase, A_CHUNK)], zsem).wait()
        return carry

    lax.fori_loop(0, A_ZITERS, zdrain, 0)
    plsc.subcore_barrier()
    lo = c * (N // NC)
    hi = lo + (N // NC)
    sink0 = (lo + s) * ROWW + N + lax.iota(jnp.int32, 16)
    bufs = (aidx0, aidx1)
    sems = (sem0, sem1)

    def step(g, carry):
        for b in range(2):
            i = 2 * g + b
            sink = sink0 + lax.rem(i, 15) * 16
            aidx = bufs[b]

            @pl.when(g > 0)
            def _():
                pltpu.make_async_copy(
                    on_v, a_out.at[pl.ds(zbase, EK)], sems[b]).wait()

            for j in range(EK // 16):
                r = ridx_all[pl.ds(i * EK + j * 16, 16)]
                cc = cidx_all[pl.ds(i * EK + j * 16, 16)]
                owned = (r >= lo) & (r < hi)
                aidx[pl.ds(j * 16, 16)] = jnp.where(owned, r * ROWW + cc,
                                                    sink)
            pltpu.async_copy(on_v, a_out.at[aidx], sems[b])
        return carry

    lax.fori_loop(0, E_ITERS_A // 2, step, 0)
    for b in range(2):
        pltpu.make_async_copy(on_v, a_out.at[pl.ds(zbase, EK)],
                              sems[b]).wait()


@functools.partial(
    pl.kernel,
    out_type=(_sds((B, D_OUT)), _sds((B, D_OUT)),
              _sds((B, NPAD // 128, 128))),
    mesh=_mesh(),
    scratch_types=[
        pltpu.VMEM((Q_PER_TILE,), jnp.int32),
        pltpu.VMEM((Q_PER_TILE,), jnp.int32),
        pltpu.VMEM((2 * Q_PER_TILE,), jnp.int32),
        pltpu.VMEM((Q_PER_TILE, D_OUT), jnp.float32),
        pltpu.VMEM((8, ROWW), jnp.float32),
        pltpu.VMEM((NPAD // 128, 128), jnp.float32),
        pltpu.SemaphoreType.DMA,
    ],
)
def _sc_gm(hn_hbm, a2_hbm, u_hbm, v_hbm, uvi_hbm,
           hu_out, hv_out, m_out,
           uq, vq, uvq, hnr, auv, mrow, sem):
    c = lax.axis_index("c")
    s = lax.axis_index("s")
    wid = s * NC + c
    qb = wid * Q_PER_TILE
    pltpu.sync_copy(u_hbm.at[pl.ds(qb, Q_PER_TILE)], uq)
    pltpu.async_copy(hn_hbm.at[uq], hnr, sem).wait()
    pltpu.sync_copy(hnr, hu_out.at[pl.ds(qb, Q_PER_TILE)])
    pltpu.sync_copy(v_hbm.at[pl.ds(qb, Q_PER_TILE)], vq)
    pltpu.async_copy(hn_hbm.at[vq], hnr, sem).wait()
    pltpu.sync_copy(hnr, hv_out.at[pl.ds(qb, Q_PER_TILE)])
    pltpu.sync_copy(uvi_hbm.at[pl.ds(2 * qb, 2 * Q_PER_TILE)], uvq)
    z16 = jnp.zeros((16,), jnp.float32)
    for j in range(N // 16, NPAD // 16):
        mrow[j // 8, pl.ds((j % 8) * 16, 16)] = z16

    def qstep(g, carry):
        pltpu.async_copy(a2_hbm.at[uvq.at[pl.ds(8 * g, 8)]], auv, sem).wait()
        for i in range(4):
            def colstep(j, carry2, _i=i):
                st = j * 16
                mrow[j // 8, pl.ds((j % 8) * 16, 16)] = (
                    auv[2 * _i, pl.ds(st, 16)] *
                    auv[2 * _i + 1, pl.ds(st, 16)])
                return carry2

            lax.fori_loop(0, N // 16, colstep, 0)
            pltpu.sync_copy(mrow, m_out.at[qb + 4 * g + i])
        return carry

    lax.fori_loop(0, Q_PER_TILE // 4, qstep, 0)


def _tc_mlp_body(x_ref, agg_ref, deg_ref, w1_ref, b1_ref, w2_ref, b2_ref,
                 w3_ref, b3_ref, hn_ref):
    deg = deg_ref[0] + deg_ref[1] + 1e-6
    agg = agg_ref[0] + agg_ref[1]
    h = x_ref[...] + agg / deg
    h = jnp.maximum(jnp.dot(h, w1_ref[...],
                            preferred_element_type=jnp.float32)
                    + b1_ref[...], 0.0)
    h = jnp.maximum(jnp.dot(h, w2_ref[...],
                            preferred_element_type=jnp.float32)
                    + b2_ref[...], 0.0)
    h = jnp.dot(h, w3_ref[...], preferred_element_type=jnp.float32) \
        + b3_ref[...]
    nrm = jnp.sqrt(jnp.sum(h * h, axis=1, keepdims=True))
    hn_ref[...] = h / jnp.maximum(nrm, 1e-8)


def _tc_mlp(xp, agg2, deg3, W1, b1, W2, b2, W3, b3):
    g = NPAD // RB
    return pl.pallas_call(
        _tc_mlp_body,
        grid=(g,),
        in_specs=[
            pl.BlockSpec((RB, D_IN), lambda i: (i, 0)),
            pl.BlockSpec((NC, RB, D_IN), lambda i: (0, i, 0)),
            pl.BlockSpec((NC, RB, 1), lambda i: (0, i, 0)),
            pl.BlockSpec((D_IN, D_HID), lambda i: (0, 0)),
            pl.BlockSpec((1, D_HID), lambda i: (0, 0)),
            pl.BlockSpec((D_HID, D_HID), lambda i: (0, 0)),
            pl.BlockSpec((1, D_HID), lambda i: (0, 0)),
            pl.BlockSpec((D_HID, D_OUT), lambda i: (0, 0)),
            pl.BlockSpec((1, D_OUT), lambda i: (0, 0)),
        ],
        out_specs=pl.BlockSpec((RB, D_OUT), lambda i: (i, 0)),
        out_shape=_sds((NPAD, D_OUT)),
    )(xp, agg2, deg3, W1, b1, W2, b2, W3, b3)


def _tc_query_body(hu_ref, hv_ref, hn_ref, m_ref, w_ref, acc):
    i = pl.program_id(0)

    @pl.when(i == 0)
    def _():
        acc[...] = jnp.zeros_like(acc)

    hnb = hn_ref[...]
    cl = lax.dot_general(hu_ref[...], hnb, (((1,), (1,)), ((), ())),
                         preferred_element_type=jnp.float32)
    cr = lax.dot_general(hv_ref[...], hnb, (((1,), (1,)), ((), ())),
                         preferred_element_type=jnp.float32)
    acc[...] += jnp.sum(m_ref[...] * cl * cr, axis=1, keepdims=True)

    @pl.when(i == pl.num_programs(0) - 1)
    def _():
        w_ref[...] = jax.nn.sigmoid(acc[...])


def _tc_query(hu, hv, hn, M):
    g = NPAD // RB
    return pl.pallas_call(
        _tc_query_body,
        grid=(g,),
        in_specs=[
            pl.BlockSpec((B, D_OUT), lambda i: (0, 0)),
            pl.BlockSpec((B, D_OUT), lambda i: (0, 0)),
            pl.BlockSpec((RB, D_OUT), lambda i: (i, 0)),
            pl.BlockSpec((B, RB), lambda i: (0, i)),
        ],
        out_specs=pl.BlockSpec((B, 1), lambda i: (0, 0)),
        out_shape=_sds((B, 1)),
        scratch_shapes=[pltpu.VMEM((B, 1), jnp.float32)],
    )(hu, hv, hn, M)


def kernel(x, edges, adj, W1, b1, W2, b2, W3, b3):
    row = adj[0].astype(jnp.int32)
    col = adj[1].astype(jnp.int32)
    u = edges[0].astype(jnp.int32)
    v = edges[1].astype(jnp.int32)

    zb = jnp.zeros((128, D_IN), jnp.float32)
    zd = jnp.zeros((1024,), jnp.float32)
    on = jnp.ones((EK,), jnp.float32)
    zf = jnp.zeros((A_CHUNK,), jnp.float32)
    uvi = jnp.stack([u, v], axis=1).reshape(-1)

    agg2, deg2 = _sc_agg(x, row, col, zb, zd, on)
    aflat = _sc_abuild(row, col, zf, on)
    xp = jnp.pad(x, ((0, NPAD - N), (0, 0)))
    deg3 = deg2.reshape(NC, NPAD, 1)
    hn = _tc_mlp(xp, agg2, deg3, W1, b1.reshape(1, D_HID),
                 W2, b2.reshape(1, D_HID), W3, b3.reshape(1, D_OUT))
    a2 = aflat.reshape(N, ROWW)
    hu, hv, M3 = _sc_gm(hn, a2, u, v, uvi)
    w = _tc_query(hu, hv, hn, M3.reshape(B, NPAD))
    return w.reshape(B)


# speedup vs baseline: 12.3985x; 1.5376x over previous
import functools

import jax
import jax.numpy as jnp
from jax import lax
from jax.experimental import pallas as pl
from jax.experimental.pallas import tpu as pltpu
from jax.experimental.pallas import tpu_sc as plsc

N = 10000
E = 320000
B = 2048
D_IN = 128
D_HID = 256
D_OUT = 128

NC = 2
NS = 16
NW = NC * NS

NPAD = 10240
RB = 1024

ROWW = NPAD
A_SIZE = N * ROWW
A_PER_CORE = A_SIZE // NC
A_PER_TILE = A_PER_CORE // NS
A_CHUNK = 32000
A_ZITERS = A_PER_TILE // A_CHUNK
SK = 128

EK = 80
E_PER_TILE_AGG = E // NW
E_ITERS_AGG = E_PER_TILE_AGG // EK
E_PER_TILE_A = E // NS
E_ITERS_A = E_PER_TILE_A // EK

Q_PER_TILE = B // NW

_mesh = functools.partial(
    plsc.VectorSubcoreMesh,
    core_axis_name="c", subcore_axis_name="s", num_cores=NC, num_subcores=NS)


def _sds(shape, dtype=jnp.float32):
    return jax.ShapeDtypeStruct(shape, dtype)


@functools.partial(
    pl.kernel,
    out_type=(_sds((NC, NPAD, D_IN)), _sds((NC, NPAD))),
    mesh=_mesh(),
    scratch_types=[
        pltpu.VMEM_SHARED((NPAD, D_IN), jnp.float32),
        pltpu.VMEM_SHARED((NPAD,), jnp.float32),
        pltpu.VMEM((EK,), jnp.int32),
        pltpu.VMEM((EK,), jnp.int32),
        pltpu.VMEM((EK, D_IN), jnp.float32),
        pltpu.VMEM((128, D_IN), jnp.float32),
        pltpu.VMEM((1024,), jnp.float32),
        pltpu.VMEM((EK,), jnp.float32),
        pltpu.SemaphoreType.DMA,
    ],
)
def _sc_agg(x_hbm, row_hbm, col_hbm, zb_hbm, zd_hbm, on_hbm,
            agg_out, deg_out,
            agg_sh, deg_sh, ridx, cidx, rows, zb_v, zd_v, on_v, sem):
    c = lax.axis_index("c")
    s = lax.axis_index("s")
    wid = s * NC + c
    pltpu.sync_copy(zb_hbm, zb_v)
    pltpu.sync_copy(zd_hbm, zd_v)
    pltpu.sync_copy(on_hbm.at[pl.ds(0, EK)], on_v)
    for k in range(5):
        pltpu.sync_copy(zb_v, agg_sh.at[pl.ds(s * 640 + k * 128, 128)])

    @pl.when(s < 10)
    def _():
        pltpu.sync_copy(zd_v, deg_sh.at[pl.ds(s * 1024, 1024)])

    plsc.subcore_barrier()
    base = wid * E_PER_TILE_AGG

    def step(i, carry):
        off = base + i * EK
        pltpu.sync_copy(row_hbm.at[pl.ds(off, EK)], ridx)
        pltpu.sync_copy(col_hbm.at[pl.ds(off, EK)], cidx)
        pltpu.async_copy(x_hbm.at[cidx], rows, sem).wait()
        pltpu.sync_copy(rows, agg_sh.at[ridx], add=True)
        pltpu.sync_copy(on_v, deg_sh.at[ridx], add=True)
        return carry

    lax.fori_loop(0, E_ITERS_AGG, step, 0)
    plsc.subcore_barrier()
    pltpu.sync_copy(agg_sh.at[pl.ds(s * 640, 640)],
                    agg_out.at[c, pl.ds(s * 640, 640)])

    @pl.when(s == 0)
    def _():
        pltpu.sync_copy(deg_sh, deg_out.at[c])


@functools.partial(
    pl.kernel,
    out_type=_sds((A_SIZE,)),
    mesh=_mesh(),
    scratch_types=[
        pltpu.VMEM((EK,), jnp.int32),
        pltpu.VMEM((EK,), jnp.int32),
        pltpu.VMEM((2 * EK,), jnp.int32),
        pltpu.VMEM((A_CHUNK,), jnp.float32),
        pltpu.VMEM((EK,), jnp.float32),
        pltpu.SemaphoreType.DMA,
        pltpu.SemaphoreType.DMA,
    ],
)
def _sc_abuild(row_hbm, col_hbm, zf_hbm, on_hbm,
               a_out, ridx, cidx, fidx, zf_v, on_v, zsem, ssem):
    c = lax.axis_index("c")
    s = lax.axis_index("s")
    ebase = s * E_PER_TILE_A
    pltpu.sync_copy(zf_hbm, zf_v)
    pltpu.sync_copy(on_hbm.at[pl.ds(0, EK)], on_v)
    zbase = c * A_PER_CORE + s * A_PER_TILE

    def zstep(i, carry):
        pltpu.async_copy(zf_v, a_out.at[pl.ds(zbase + i * A_CHUNK, A_CHUNK)],
                         zsem)
        return carry

    lax.fori_loop(0, A_ZITERS, zstep, 0)

    def zdrain(i, carry):
        pltpu.make_async_copy(
            zf_v, a_out.at[pl.ds(zbase, A_CHUNK)], zsem).wait()
        return carry

    lax.fori_loop(0, A_ZITERS, zdrain, 0)
    plsc.subcore_barrier()
    lo = c * (N // NC)
    hi = lo + (N // NC)
    lanes = lax.iota(jnp.int32, 16)
    nsub = EK // 16
    srows = (N // NC) // NS

    def estep(i, carry):
        buf = (i % 2) * EK
        off = ebase + i * EK
        pltpu.sync_copy(row_hbm.at[pl.ds(off, EK)], ridx)
        pltpu.sync_copy(col_hbm.at[pl.ds(off, EK)], cidx)

        @pl.when(i >= 1)
        def _():
            pltpu.make_async_copy(
                on_v, a_out.at[pl.ds(zbase, EK)], ssem).wait()

        for j in range(nsub):
            r = ridx[pl.ds(j * 16, 16)]
            cc = cidx[pl.ds(j * 16, 16)]
            owned = (r >= lo) & (r < hi)
            srow = lo + s * srows + (i * nsub + j) % srows
            sink = srow * ROWW + N + j * 16 + lanes
            fidx[pl.ds(buf + j * 16, 16)] = jnp.where(
                owned, r * ROWW + cc, sink)
        pltpu.async_copy(on_v, a_out.at[fidx.at[pl.ds(buf, EK)]], ssem)
        return carry

    lax.fori_loop(0, E_ITERS_A, estep, 0)
    pltpu.make_async_copy(on_v, a_out.at[pl.ds(zbase, EK)], ssem).wait()


@functools.partial(
    pl.kernel,
    out_type=(_sds((B, D_OUT)), _sds((B, D_OUT)),
              _sds((B, NPAD // 128, 128))),
    mesh=_mesh(),
    scratch_types=[
        pltpu.VMEM((Q_PER_TILE,), jnp.int32),
        pltpu.VMEM((Q_PER_TILE,), jnp.int32),
        pltpu.VMEM((2 * Q_PER_TILE,), jnp.int32),
        pltpu.VMEM((Q_PER_TILE, D_OUT), jnp.float32),
        pltpu.VMEM((8, ROWW), jnp.float32),
        pltpu.VMEM((NPAD // 128, 128), jnp.float32),
        pltpu.SemaphoreType.DMA,
    ],
)
def _sc_gm(hn_hbm, a2_hbm, u_hbm, v_hbm, uvi_hbm,
           hu_out, hv_out, m_out,
           uq, vq, uvq, hnr, auv, mrow, sem):
    c = lax.axis_index("c")
    s = lax.axis_index("s")
    wid = s * NC + c
    qb = wid * Q_PER_TILE
    pltpu.sync_copy(u_hbm.at[pl.ds(qb, Q_PER_TILE)], uq)
    pltpu.async_copy(hn_hbm.at[uq], hnr, sem).wait()
    pltpu.sync_copy(hnr, hu_out.at[pl.ds(qb, Q_PER_TILE)])
    pltpu.sync_copy(v_hbm.at[pl.ds(qb, Q_PER_TILE)], vq)
    pltpu.async_copy(hn_hbm.at[vq], hnr, sem).wait()
    pltpu.sync_copy(hnr, hv_out.at[pl.ds(qb, Q_PER_TILE)])
    pltpu.sync_copy(uvi_hbm.at[pl.ds(2 * qb, 2 * Q_PER_TILE)], uvq)
    z16 = jnp.zeros((16,), jnp.float32)
    for j in range(N // 16, NPAD // 16):
        mrow[j // 8, pl.ds((j % 8) * 16, 16)] = z16

    def qstep(g, carry):
        pltpu.async_copy(a2_hbm.at[uvq.at[pl.ds(8 * g, 8)]], auv, sem).wait()
        for i in range(4):
            def colstep(j, carry2, _i=i):
                st = j * 16
                mrow[j // 8, pl.ds((j % 8) * 16, 16)] = (
                    auv[2 * _i, pl.ds(st, 16)] *
                    auv[2 * _i + 1, pl.ds(st, 16)])
                return carry2

            lax.fori_loop(0, N // 16, colstep, 0)
            pltpu.sync_copy(mrow, m_out.at[qb + 4 * g + i])
        return carry

    lax.fori_loop(0, Q_PER_TILE // 4, qstep, 0)


def _tc_mlp_body(x_ref, agg_ref, deg_ref, w1_ref, b1_ref, w2_ref, b2_ref,
                 w3_ref, b3_ref, hn_ref):
    deg = deg_ref[0] + deg_ref[1] + 1e-6
    agg = agg_ref[0] + agg_ref[1]
    h = x_ref[...] + agg / deg
    h = jnp.maximum(jnp.dot(h, w1_ref[...],
                            preferred_element_type=jnp.float32)
                    + b1_ref[...], 0.0)
    h = jnp.maximum(jnp.dot(h, w2_ref[...],
                            preferred_element_type=jnp.float32)
                    + b2_ref[...], 0.0)
    h = jnp.dot(h, w3_ref[...], preferred_element_type=jnp.float32) \
        + b3_ref[...]
    nrm = jnp.sqrt(jnp.sum(h * h, axis=1, keepdims=True))
    hn_ref[...] = h / jnp.maximum(nrm, 1e-8)


def _tc_mlp(xp, agg2, deg3, W1, b1, W2, b2, W3, b3):
    g = NPAD // RB
    return pl.pallas_call(
        _tc_mlp_body,
        grid=(g,),
        in_specs=[
            pl.BlockSpec((RB, D_IN), lambda i: (i, 0)),
            pl.BlockSpec((NC, RB, D_IN), lambda i: (0, i, 0)),
            pl.BlockSpec((NC, RB, 1), lambda i: (0, i, 0)),
            pl.BlockSpec((D_IN, D_HID), lambda i: (0, 0)),
            pl.BlockSpec((1, D_HID), lambda i: (0, 0)),
            pl.BlockSpec((D_HID, D_HID), lambda i: (0, 0)),
            pl.BlockSpec((1, D_HID), lambda i: (0, 0)),
            pl.BlockSpec((D_HID, D_OUT), lambda i: (0, 0)),
            pl.BlockSpec((1, D_OUT), lambda i: (0, 0)),
        ],
        out_specs=pl.BlockSpec((RB, D_OUT), lambda i: (i, 0)),
        out_shape=_sds((NPAD, D_OUT)),
    )(xp, agg2, deg3, W1, b1, W2, b2, W3, b3)


def _tc_query_body(hu_ref, hv_ref, hn_ref, m_ref, w_ref, acc):
    i = pl.program_id(0)

    @pl.when(i == 0)
    def _():
        acc[...] = jnp.zeros_like(acc)

    hnb = hn_ref[...]
    cl = lax.dot_general(hu_ref[...], hnb, (((1,), (1,)), ((), ())),
                         preferred_element_type=jnp.float32)
    cr = lax.dot_general(hv_ref[...], hnb, (((1,), (1,)), ((), ())),
                         preferred_element_type=jnp.float32)
    acc[...] += jnp.sum(m_ref[...] * cl * cr, axis=1, keepdims=True)

    @pl.when(i == pl.num_programs(0) - 1)
    def _():
        w_ref[...] = jax.nn.sigmoid(acc[...])


def _tc_query(hu, hv, hn, M):
    g = NPAD // RB
    return pl.pallas_call(
        _tc_query_body,
        grid=(g,),
        in_specs=[
            pl.BlockSpec((B, D_OUT), lambda i: (0, 0)),
            pl.BlockSpec((B, D_OUT), lambda i: (0, 0)),
            pl.BlockSpec((RB, D_OUT), lambda i: (i, 0)),
            pl.BlockSpec((B, RB), lambda i: (0, i)),
        ],
        out_specs=pl.BlockSpec((B, 1), lambda i: (0, 0)),
        out_shape=_sds((B, 1)),
        scratch_shapes=[pltpu.VMEM((B, 1), jnp.float32)],
    )(hu, hv, hn, M)


def kernel(x, edges, adj, W1, b1, W2, b2, W3, b3):
    row = adj[0].astype(jnp.int32)
    col = adj[1].astype(jnp.int32)
    u = edges[0].astype(jnp.int32)
    v = edges[1].astype(jnp.int32)

    zb = jnp.zeros((128, D_IN), jnp.float32)
    zd = jnp.zeros((1024,), jnp.float32)
    on = jnp.ones((SK,), jnp.float32)
    zf = jnp.zeros((A_CHUNK,), jnp.float32)
    uvi = jnp.stack([u, v], axis=1).reshape(-1)

    agg2, deg2 = _sc_agg(x, row, col, zb, zd, on)
    aflat = _sc_abuild(row, col, zf, on)
    xp = jnp.pad(x, ((0, NPAD - N), (0, 0)))
    deg3 = deg2.reshape(NC, NPAD, 1)
    hn = _tc_mlp(xp, agg2, deg3, W1, b1.reshape(1, D_HID),
                 W2, b2.reshape(1, D_HID), W3, b3.reshape(1, D_OUT))
    a2 = aflat.reshape(N, ROWW)
    hu, hv, M3 = _sc_gm(hn, a2, u, v, uvi)
    w = _tc_query(hu, hv, hn, M3.reshape(B, NPAD))
    return w.reshape(B)
